# Initial kernel scaffold; baseline (speedup 1.0000x reference)
import jax
import jax.numpy as jnp
from jax.experimental import pallas as pl


def kernel(xyz, sparse_xyz, sparse_flow):
    raise NotImplementedError("write your pallas kernel here")


# speedup vs baseline: 28.1884x; 28.1884x over previous
import functools

import jax
import jax.numpy as jnp
from jax import lax
from jax.experimental import pallas as pl
from jax.experimental.pallas import tpu as pltpu

_N_TILE = 512
_BIG = 3.4e38


def _upsample_kernel(x_ref, sx_ref, flow_ref, out_ref, *, S):
    x0 = x_ref[:, 0:1]
    x1 = x_ref[:, 1:2]
    x2c = x_ref[:, 2:3]
    s0 = sx_ref[0:1, :]
    s1 = sx_ref[1:2, :]
    s2c = sx_ref[2:3, :]
    xsq = x0 * x0 + x1 * x1 + x2c * x2c
    ssq = s0 * s0 + s1 * s1 + s2c * s2c
    inner = jnp.dot(x_ref[...], sx_ref[...],
                    preferred_element_type=jnp.float32)
    d2 = jnp.maximum(xsq + ssq - 2.0 * inner, 0.0)

    T = d2.shape[0]
    iota = lax.broadcasted_iota(jnp.int32, (T, S), 1)
    f0 = flow_ref[0:1, :]
    f1 = flow_ref[1:2, :]
    f2 = flow_ref[2:3, :]

    acc0 = jnp.zeros((T, 1), jnp.float32)
    acc1 = jnp.zeros((T, 1), jnp.float32)
    acc2 = jnp.zeros((T, 1), jnp.float32)
    for _ in range(3):
        m = jnp.min(d2, axis=1, keepdims=True)
        cand = jnp.where(d2 == m, iota, S)
        amin = jnp.min(cand, axis=1, keepdims=True)
        onehot = iota == amin
        d2 = jnp.where(onehot, _BIG, d2)
        dist = jnp.sqrt(jnp.maximum(m, 1e-20))
        w = 1.0 / jnp.maximum(dist, 1e-10)
        acc0 += w * jnp.sum(jnp.where(onehot, f0, 0.0), axis=1, keepdims=True)
        acc1 += w * jnp.sum(jnp.where(onehot, f1, 0.0), axis=1, keepdims=True)
        acc2 += w * jnp.sum(jnp.where(onehot, f2, 0.0), axis=1, keepdims=True)
    out_ref[:, 0:1] = acc0
    out_ref[:, 1:2] = acc1
    out_ref[:, 2:3] = acc2


def kernel(xyz, sparse_xyz, sparse_flow):
    B, _, N = xyz.shape
    S = sparse_xyz.shape[2]
    C = sparse_flow.shape[1]
    x_t = jnp.transpose(xyz, (0, 2, 1))
    grid = (B, N // _N_TILE)
    out = pl.pallas_call(
        functools.partial(_upsample_kernel, S=S),
        grid=grid,
        in_specs=[
            pl.BlockSpec((None, _N_TILE, 3), lambda b, n: (b, n, 0)),
            pl.BlockSpec((None, 3, S), lambda b, n: (b, 0, 0)),
            pl.BlockSpec((None, C, S), lambda b, n: (b, 0, 0)),
        ],
        out_specs=pl.BlockSpec((None, _N_TILE, C), lambda b, n: (b, n, 0)),
        out_shape=jax.ShapeDtypeStruct((B, N, C), jnp.float32),
    )(x_t, sparse_xyz, sparse_flow)
    return jnp.transpose(out, (0, 2, 1))
